# baseline (speedup 1.0000x reference)
import functools

import jax
import jax.numpy as jnp
from jax import lax
from jax.experimental import pallas as pl
from jax.experimental.pallas import tpu as pltpu
from jax.experimental.pallas import tpu_sc as plsc

VOCAB = 1000000
D = 64
B = 4096
L = 200
NC = 2
NS = 16
NW = NC * NS
PER_W = B // NW
L_LO = 104
L_HI = L - L_LO
LANES = 16
NCH = D // LANES

TBLK = 2048
NBLK = -(-VOCAB // TBLK)
ROWS = NBLK * (TBLK // 2)
VOCAB_PAD = 2 * ROWS


def _repack_tc(emb_t):
    def kern(in_ref, o_ref):
        for s in range(TBLK // 1024):
            lft = in_ref[:, pl.ds(1024 * s, 512)]
            rgt = in_ref[:, pl.ds(1024 * s + 512, 512)]
            o_ref[pl.ds(512 * s, 512), :] = jnp.transpose(
                jnp.concatenate([lft, rgt], axis=0))

    return pl.pallas_call(
        kern,
        grid=(NBLK,),
        in_specs=[pl.BlockSpec((D, TBLK), lambda i: (0, i))],
        out_specs=pl.BlockSpec((TBLK // 2, 2 * D), lambda i: (i, 0)),
        out_shape=jax.ShapeDtypeStruct((ROWS, 2 * D), jnp.float32),
    )(emb_t)


def _remap_idx(x):
    v = x.astype(jnp.int32)
    v = (v & jnp.int32(~1023)) + ((v & 511) << 1) + ((v >> 9) & 1)
    return v.reshape(B * L)


def _pooled_sc(x_flat, table):
    mesh = plsc.VectorSubcoreMesh(core_axis_name="c", subcore_axis_name="s")

    @functools.partial(
        pl.kernel,
        out_type=jax.ShapeDtypeStruct((B, D), jnp.float32),
        mesh=mesh,
        scratch_types=[
            pltpu.VMEM((PER_W * L,), jnp.int32),
            pltpu.VMEM((L, D), jnp.float32),
            pltpu.VMEM((L, D), jnp.float32),
            pltpu.VMEM((PER_W, D), jnp.float32),
            pltpu.SemaphoreType.DMA,
            pltpu.SemaphoreType.DMA,
        ],
        compiler_params=pltpu.CompilerParams(use_tc_tiling_on_sc=False),
    )
    def kern(x_hbm, tab_hbm, out_hbm, idx_v, rows0, rows1, out_v, sem0, sem1):
        cid = lax.axis_index("c")
        sid = lax.axis_index("s")
        wid = sid * NC + cid
        base = pl.multiple_of(wid * PER_W, PER_W)

        pltpu.sync_copy(
            x_hbm.at[pl.ds(pl.multiple_of(wid * (PER_W * L), 8), PER_W * L)],
            idx_v)

        scale = jnp.float32(1.0 / L)

        def issue(i, buf, sem):
            off = pl.multiple_of(i * L, 8)
            pltpu.async_copy(
                tab_hbm.at[idx_v.at[pl.ds(off, L_LO)]],
                buf.at[pl.ds(0, L_LO)], sem)
            pltpu.async_copy(
                tab_hbm.at[idx_v.at[pl.ds(off + L_LO, L_HI)]],
                buf.at[pl.ds(L_LO, L_HI)], sem)

        def wait(buf, sem):
            pltpu.make_async_copy(tab_hbm.at[pl.ds(0, L)], buf, sem).wait()

        def accum(buf, i):
            zeros = (jnp.zeros((LANES,), jnp.float32),) * NCH

            @pl.loop(0, L, init_carry=zeros, unroll=8)
            def red(r, acc):
                return tuple(
                    acc[j] + buf[r, pl.ds(j * LANES, LANES)]
                    for j in range(NCH)
                )

            for j in range(NCH):
                out_v[i, pl.ds(j * LANES, LANES)] = red[j] * scale

        issue(0, rows0, sem0)

        @pl.loop(0, PER_W, step=2)
        def elem(i):
            issue(i + 1, rows1, sem1)
            wait(rows0, sem0)
            accum(rows0, i)

            @pl.when(i + 2 < PER_W)
            def _():
                issue(i + 2, rows0, sem0)

            wait(rows1, sem1)
            accum(rows1, i + 1)

        pltpu.sync_copy(out_v, out_hbm.at[pl.ds(base, PER_W)])

    return kern(x_flat, table)


def _linear_tc(pooled, w, b2):
    def kern(p_ref, w_ref, b_ref, o_ref):
        o_ref[...] = lax.dot_general(
            p_ref[...], w_ref[...], (((1,), (1,)), ((), ())),
            preferred_element_type=jnp.float32) + b_ref[...]

    return pl.pallas_call(
        kern,
        out_shape=jax.ShapeDtypeStruct((B, 2), jnp.float32),
    )(pooled, w, b2)


def kernel(x, embedding, fc_w, fc_b):
    table = _repack_tc(embedding.T).reshape(VOCAB_PAD, D)
    pooled = _pooled_sc(_remap_idx(x), table)
    return _linear_tc(pooled, fc_w, fc_b.reshape(1, 2))


# speedup vs baseline: 1.5456x; 1.5456x over previous
import functools

import jax
import jax.numpy as jnp
from jax import lax
from jax.experimental import pallas as pl
from jax.experimental.pallas import tpu as pltpu
from jax.experimental.pallas import tpu_sc as plsc

VOCAB = 1000000
D = 64
B = 4096
L = 200
NC = 2
NS = 16
NW = NC * NS
PER_W = B // NW
L_LO = 104
L_HI = L - L_LO
LANES = 16
NCH = D // LANES

TBLK = 8192
NBLK = -(-VOCAB // TBLK)
ROWS = NBLK * (TBLK // 2)
VOCAB_PAD = 2 * ROWS


def _repack_tc(emb_t):
    def kern(in_ref, o_ref):
        for s in range(TBLK // 1024):
            lft = in_ref[:, pl.ds(1024 * s, 512)]
            rgt = in_ref[:, pl.ds(1024 * s + 512, 512)]
            o_ref[pl.ds(512 * s, 512), :] = jnp.transpose(
                jnp.concatenate([lft, rgt], axis=0))

    return pl.pallas_call(
        kern,
        grid=(NBLK,),
        in_specs=[pl.BlockSpec((D, TBLK), lambda i: (0, i))],
        out_specs=pl.BlockSpec((TBLK // 2, 2 * D), lambda i: (i, 0)),
        out_shape=jax.ShapeDtypeStruct((ROWS, 2 * D), jnp.float32),
    )(emb_t)


def _remap_idx(x):
    v = x.astype(jnp.int32)
    v = (v & jnp.int32(~1023)) + ((v & 511) << 1) + ((v >> 9) & 1)
    return v.reshape(B * L)


def _pooled_sc(x_flat, table):
    mesh = plsc.VectorSubcoreMesh(core_axis_name="c", subcore_axis_name="s")

    @functools.partial(
        pl.kernel,
        out_type=jax.ShapeDtypeStruct((B, D), jnp.float32),
        mesh=mesh,
        scratch_types=[
            pltpu.VMEM((PER_W * L,), jnp.int32),
            pltpu.VMEM((L, D), jnp.float32),
            pltpu.VMEM((L, D), jnp.float32),
            pltpu.VMEM((PER_W, D), jnp.float32),
            pltpu.SemaphoreType.DMA,
            pltpu.SemaphoreType.DMA,
        ],
        compiler_params=pltpu.CompilerParams(use_tc_tiling_on_sc=False),
    )
    def kern(x_hbm, tab_hbm, out_hbm, idx_v, rows0, rows1, out_v, sem0, sem1):
        cid = lax.axis_index("c")
        sid = lax.axis_index("s")
        wid = sid * NC + cid
        base = pl.multiple_of(wid * PER_W, PER_W)

        pltpu.sync_copy(
            x_hbm.at[pl.ds(pl.multiple_of(wid * (PER_W * L), 8), PER_W * L)],
            idx_v)

        scale = jnp.float32(1.0 / L)

        def issue(i, buf, sem):
            off = pl.multiple_of(i * L, 8)
            pltpu.async_copy(
                tab_hbm.at[idx_v.at[pl.ds(off, L_LO)]],
                buf.at[pl.ds(0, L_LO)], sem)
            pltpu.async_copy(
                tab_hbm.at[idx_v.at[pl.ds(off + L_LO, L_HI)]],
                buf.at[pl.ds(L_LO, L_HI)], sem)

        def wait(buf, sem):
            pltpu.make_async_copy(tab_hbm.at[pl.ds(0, L)], buf, sem).wait()

        def accum(buf, i):
            zeros = (jnp.zeros((LANES,), jnp.float32),) * NCH

            @pl.loop(0, L, init_carry=zeros, unroll=8)
            def red(r, acc):
                return tuple(
                    acc[j] + buf[r, pl.ds(j * LANES, LANES)]
                    for j in range(NCH)
                )

            for j in range(NCH):
                out_v[i, pl.ds(j * LANES, LANES)] = red[j] * scale

        issue(0, rows0, sem0)

        @pl.loop(0, PER_W, step=2)
        def elem(i):
            issue(i + 1, rows1, sem1)
            wait(rows0, sem0)
            accum(rows0, i)

            @pl.when(i + 2 < PER_W)
            def _():
                issue(i + 2, rows0, sem0)

            wait(rows1, sem1)
            accum(rows1, i + 1)

        pltpu.sync_copy(out_v, out_hbm.at[pl.ds(base, PER_W)])

    return kern(x_flat, table)


def _linear_tc(pooled, w, b2):
    def kern(p_ref, w_ref, b_ref, o_ref):
        o_ref[...] = lax.dot_general(
            p_ref[...], w_ref[...], (((1,), (1,)), ((), ())),
            preferred_element_type=jnp.float32) + b_ref[...]

    return pl.pallas_call(
        kern,
        out_shape=jax.ShapeDtypeStruct((B, 2), jnp.float32),
    )(pooled, w, b2)


def kernel(x, embedding, fc_w, fc_b):
    table = _repack_tc(embedding.T).reshape(VOCAB_PAD, D)
    pooled = _pooled_sc(_remap_idx(x), table)
    return _linear_tc(pooled, fc_w, fc_b.reshape(1, 2))


# speedup vs baseline: 1.6789x; 1.0862x over previous
import functools

import jax
import jax.numpy as jnp
from jax import lax
from jax.experimental import pallas as pl
from jax.experimental.pallas import tpu as pltpu
from jax.experimental.pallas import tpu_sc as plsc

VOCAB = 1000000
D = 64
B = 4096
L = 200
NC = 2
NS = 16
NW = NC * NS
PER_W = B // NW
L_LO = 104
L_HI = L - L_LO
LANES = 16
NCH = D // LANES

TBLK = 16384
NBLK = -(-VOCAB // TBLK)
ROWS = NBLK * (TBLK // 2)
VOCAB_PAD = 2 * ROWS


def _repack_tc(emb_t):
    def kern(in_ref, o_ref):
        for s in range(TBLK // 1024):
            lft = in_ref[:, pl.ds(1024 * s, 512)]
            rgt = in_ref[:, pl.ds(1024 * s + 512, 512)]
            o_ref[pl.ds(512 * s, 512), :] = jnp.transpose(
                jnp.concatenate([lft, rgt], axis=0))

    return pl.pallas_call(
        kern,
        grid=(NBLK,),
        in_specs=[pl.BlockSpec((D, TBLK), lambda i: (0, i))],
        out_specs=pl.BlockSpec((TBLK // 2, 2 * D), lambda i: (i, 0)),
        out_shape=jax.ShapeDtypeStruct((ROWS, 2 * D), jnp.float32),
    )(emb_t)


def _remap_idx(x):
    v = x.astype(jnp.int32)
    v = (v & jnp.int32(~1023)) + ((v & 511) << 1) + ((v >> 9) & 1)
    return v.reshape(B * L)


def _pooled_sc(x_flat, table):
    mesh = plsc.VectorSubcoreMesh(core_axis_name="c", subcore_axis_name="s")

    @functools.partial(
        pl.kernel,
        out_type=jax.ShapeDtypeStruct((B, D), jnp.float32),
        mesh=mesh,
        scratch_types=[
            pltpu.VMEM((PER_W * L,), jnp.int32),
            pltpu.VMEM((L, D), jnp.float32),
            pltpu.VMEM((L, D), jnp.float32),
            pltpu.VMEM((PER_W, D), jnp.float32),
            pltpu.SemaphoreType.DMA,
            pltpu.SemaphoreType.DMA,
        ],
        compiler_params=pltpu.CompilerParams(use_tc_tiling_on_sc=False),
    )
    def kern(x_hbm, tab_hbm, out_hbm, idx_v, rows0, rows1, out_v, sem0, sem1):
        cid = lax.axis_index("c")
        sid = lax.axis_index("s")
        wid = sid * NC + cid
        base = pl.multiple_of(wid * PER_W, PER_W)

        pltpu.sync_copy(
            x_hbm.at[pl.ds(pl.multiple_of(wid * (PER_W * L), 8), PER_W * L)],
            idx_v)

        scale = jnp.float32(1.0 / L)

        def issue(i, buf, sem):
            off = pl.multiple_of(i * L, 8)
            pltpu.async_copy(
                tab_hbm.at[idx_v.at[pl.ds(off, L_LO)]],
                buf.at[pl.ds(0, L_LO)], sem)
            pltpu.async_copy(
                tab_hbm.at[idx_v.at[pl.ds(off + L_LO, L_HI)]],
                buf.at[pl.ds(L_LO, L_HI)], sem)

        def wait(buf, sem):
            pltpu.make_async_copy(tab_hbm.at[pl.ds(0, L)], buf, sem).wait()

        def accum(buf, i):
            zeros = (jnp.zeros((LANES,), jnp.float32),) * NCH

            @pl.loop(0, L, init_carry=zeros, unroll=8)
            def red(r, acc):
                return tuple(
                    acc[j] + buf[r, pl.ds(j * LANES, LANES)]
                    for j in range(NCH)
                )

            for j in range(NCH):
                out_v[i, pl.ds(j * LANES, LANES)] = red[j] * scale

        issue(0, rows0, sem0)

        @pl.loop(0, PER_W, step=2)
        def elem(i):
            issue(i + 1, rows1, sem1)
            wait(rows0, sem0)
            accum(rows0, i)

            @pl.when(i + 2 < PER_W)
            def _():
                issue(i + 2, rows0, sem0)

            wait(rows1, sem1)
            accum(rows1, i + 1)

        pltpu.sync_copy(out_v, out_hbm.at[pl.ds(base, PER_W)])

    return kern(x_flat, table)


def _linear_tc(pooled, w, b2):
    def kern(p_ref, w_ref, b_ref, o_ref):
        o_ref[...] = lax.dot_general(
            p_ref[...], w_ref[...], (((1,), (1,)), ((), ())),
            preferred_element_type=jnp.float32) + b_ref[...]

    return pl.pallas_call(
        kern,
        out_shape=jax.ShapeDtypeStruct((B, 2), jnp.float32),
    )(pooled, w, b2)


def kernel(x, embedding, fc_w, fc_b):
    table = _repack_tc(embedding.T).reshape(VOCAB_PAD, D)
    pooled = _pooled_sc(_remap_idx(x), table)
    return _linear_tc(pooled, fc_w, fc_b.reshape(1, 2))


# speedup vs baseline: 1.7052x; 1.0156x over previous
import functools

import jax
import jax.numpy as jnp
from jax import lax
from jax.experimental import pallas as pl
from jax.experimental.pallas import tpu as pltpu
from jax.experimental.pallas import tpu_sc as plsc

VOCAB = 1000000
D = 64
B = 4096
L = 200
NC = 2
NS = 16
NW = NC * NS
PER_W = B // NW
L_LO = 104
L_HI = L - L_LO
LANES = 16
NCH = D // LANES

TBLK = 32768
NBLK = -(-VOCAB // TBLK)
ROWS = NBLK * (TBLK // 2)
VOCAB_PAD = 2 * ROWS


def _repack_tc(emb_t):
    def kern(in_ref, o_ref):
        for s in range(TBLK // 1024):
            lft = in_ref[:, pl.ds(1024 * s, 512)]
            rgt = in_ref[:, pl.ds(1024 * s + 512, 512)]
            o_ref[pl.ds(512 * s, 512), :] = jnp.transpose(
                jnp.concatenate([lft, rgt], axis=0))

    return pl.pallas_call(
        kern,
        grid=(NBLK,),
        in_specs=[pl.BlockSpec((D, TBLK), lambda i: (0, i))],
        out_specs=pl.BlockSpec((TBLK // 2, 2 * D), lambda i: (i, 0)),
        out_shape=jax.ShapeDtypeStruct((ROWS, 2 * D), jnp.float32),
    )(emb_t)


def _remap_idx(x):
    v = x.astype(jnp.int32)
    v = (v & jnp.int32(~1023)) + ((v & 511) << 1) + ((v >> 9) & 1)
    return v.reshape(B * L)


def _pooled_sc(x_flat, table):
    mesh = plsc.VectorSubcoreMesh(core_axis_name="c", subcore_axis_name="s")

    @functools.partial(
        pl.kernel,
        out_type=jax.ShapeDtypeStruct((B, D), jnp.float32),
        mesh=mesh,
        scratch_types=[
            pltpu.VMEM((PER_W * L,), jnp.int32),
            pltpu.VMEM((L, D), jnp.float32),
            pltpu.VMEM((L, D), jnp.float32),
            pltpu.VMEM((PER_W, D), jnp.float32),
            pltpu.SemaphoreType.DMA,
            pltpu.SemaphoreType.DMA,
        ],
        compiler_params=pltpu.CompilerParams(use_tc_tiling_on_sc=False),
    )
    def kern(x_hbm, tab_hbm, out_hbm, idx_v, rows0, rows1, out_v, sem0, sem1):
        cid = lax.axis_index("c")
        sid = lax.axis_index("s")
        wid = sid * NC + cid
        base = pl.multiple_of(wid * PER_W, PER_W)

        pltpu.sync_copy(
            x_hbm.at[pl.ds(pl.multiple_of(wid * (PER_W * L), 8), PER_W * L)],
            idx_v)

        scale = jnp.float32(1.0 / L)

        def issue(i, buf, sem):
            off = pl.multiple_of(i * L, 8)
            pltpu.async_copy(
                tab_hbm.at[idx_v.at[pl.ds(off, L_LO)]],
                buf.at[pl.ds(0, L_LO)], sem)
            pltpu.async_copy(
                tab_hbm.at[idx_v.at[pl.ds(off + L_LO, L_HI)]],
                buf.at[pl.ds(L_LO, L_HI)], sem)

        def wait(buf, sem):
            pltpu.make_async_copy(tab_hbm.at[pl.ds(0, L)], buf, sem).wait()

        def accum(buf, i):
            zeros = (jnp.zeros((LANES,), jnp.float32),) * NCH

            @pl.loop(0, L, init_carry=zeros, unroll=8)
            def red(r, acc):
                return tuple(
                    acc[j] + buf[r, pl.ds(j * LANES, LANES)]
                    for j in range(NCH)
                )

            for j in range(NCH):
                out_v[i, pl.ds(j * LANES, LANES)] = red[j] * scale

        issue(0, rows0, sem0)

        @pl.loop(0, PER_W, step=2)
        def elem(i):
            issue(i + 1, rows1, sem1)
            wait(rows0, sem0)
            accum(rows0, i)

            @pl.when(i + 2 < PER_W)
            def _():
                issue(i + 2, rows0, sem0)

            wait(rows1, sem1)
            accum(rows1, i + 1)

        pltpu.sync_copy(out_v, out_hbm.at[pl.ds(base, PER_W)])

    return kern(x_flat, table)


def _linear_tc(pooled, w, b2):
    def kern(p_ref, w_ref, b_ref, o_ref):
        o_ref[...] = lax.dot_general(
            p_ref[...], w_ref[...], (((1,), (1,)), ((), ())),
            preferred_element_type=jnp.float32) + b_ref[...]

    return pl.pallas_call(
        kern,
        out_shape=jax.ShapeDtypeStruct((B, 2), jnp.float32),
    )(pooled, w, b2)


def kernel(x, embedding, fc_w, fc_b):
    table = _repack_tc(embedding.T).reshape(VOCAB_PAD, D)
    pooled = _pooled_sc(_remap_idx(x), table)
    return _linear_tc(pooled, fc_w, fc_b.reshape(1, 2))


# speedup vs baseline: 1.9624x; 1.1509x over previous
import functools

import jax
import jax.numpy as jnp
from jax import lax
from jax.experimental import pallas as pl
from jax.experimental.pallas import tpu as pltpu
from jax.experimental.pallas import tpu_sc as plsc

VOCAB = 1000000
D = 64
B = 4096
L = 200
NC = 2
NS = 16
NW = NC * NS
PER_W = B // NW
L_LO = 104
L_HI = L - L_LO
LANES = 16
NCH = D // LANES

TBLK = 32768
NBLK = -(-VOCAB // TBLK)
ROWS = NBLK * (TBLK // 2)
VOCAB_PAD = 2 * ROWS


def _repack_tc(emb_t):
    def kern(in_ref, o_ref):
        for s in range(TBLK // 1024):
            lft = in_ref[:, pl.ds(1024 * s, 512)]
            rgt = in_ref[:, pl.ds(1024 * s + 512, 512)]
            o_ref[pl.ds(512 * s, 512), :] = jnp.transpose(
                jnp.concatenate([lft, rgt], axis=0))

    return pl.pallas_call(
        kern,
        grid=(NBLK,),
        in_specs=[pl.BlockSpec((D, TBLK), lambda i: (0, i))],
        out_specs=pl.BlockSpec((TBLK // 2, 2 * D), lambda i: (i, 0)),
        out_shape=jax.ShapeDtypeStruct((ROWS, 2 * D), jnp.float32),
    )(emb_t)


def _remap_idx(x):
    v = x.astype(jnp.int32)
    v = (v & jnp.int32(~1023)) + ((v & 511) << 1) + ((v >> 9) & 1)
    return v.reshape(B * L)


def _pooled_sc(x_flat, table):
    mesh = plsc.VectorSubcoreMesh(core_axis_name="c", subcore_axis_name="s")

    @functools.partial(
        pl.kernel,
        out_type=jax.ShapeDtypeStruct((B, D), jnp.float32),
        mesh=mesh,
        scratch_types=[
            pltpu.VMEM((PER_W * L,), jnp.int32),
            pltpu.VMEM((L, D), jnp.float32),
            pltpu.VMEM((L, D), jnp.float32),
            pltpu.VMEM((L, D), jnp.float32),
            pltpu.VMEM((L, D), jnp.float32),
            pltpu.VMEM((PER_W, D), jnp.float32),
            pltpu.SemaphoreType.DMA,
            pltpu.SemaphoreType.DMA,
            pltpu.SemaphoreType.DMA,
            pltpu.SemaphoreType.DMA,
        ],
        compiler_params=pltpu.CompilerParams(use_tc_tiling_on_sc=False),
    )
    def kern(x_hbm, tab_hbm, out_hbm, idx_v, rows0, rows1, rows2, rows3,
             out_v, sem0, sem1, sem2, sem3):
        cid = lax.axis_index("c")
        sid = lax.axis_index("s")
        wid = sid * NC + cid
        base = pl.multiple_of(wid * PER_W, PER_W)

        pltpu.sync_copy(
            x_hbm.at[pl.ds(pl.multiple_of(wid * (PER_W * L), 8), PER_W * L)],
            idx_v)

        scale = jnp.float32(1.0 / L)

        def issue(i, buf, sem):
            off = pl.multiple_of(i * L, 8)
            pltpu.async_copy(
                tab_hbm.at[idx_v.at[pl.ds(off, L_LO)]],
                buf.at[pl.ds(0, L_LO)], sem)
            pltpu.async_copy(
                tab_hbm.at[idx_v.at[pl.ds(off + L_LO, L_HI)]],
                buf.at[pl.ds(L_LO, L_HI)], sem)

        def wait(buf, sem):
            pltpu.make_async_copy(tab_hbm.at[pl.ds(0, L)], buf, sem).wait()

        def accum(buf, i):
            zeros = (jnp.zeros((LANES,), jnp.float32),) * NCH

            @pl.loop(0, L, init_carry=zeros, unroll=8)
            def red(r, acc):
                return tuple(
                    acc[j] + buf[r, pl.ds(j * LANES, LANES)]
                    for j in range(NCH)
                )

            for j in range(NCH):
                out_v[i, pl.ds(j * LANES, LANES)] = red[j] * scale

        bufs = (rows0, rows1, rows2, rows3)
        sems = (sem0, sem1, sem2, sem3)
        NBUF = 4

        for k in range(NBUF - 1):
            issue(k, bufs[k], sems[k])

        @pl.loop(0, PER_W, step=NBUF)
        def elem(i):
            for k in range(NBUF):
                nxt = i + k + NBUF - 1

                @pl.when(nxt < PER_W)
                def _():
                    issue(nxt, bufs[(k + NBUF - 1) % NBUF],
                          sems[(k + NBUF - 1) % NBUF])

                wait(bufs[k], sems[k])
                accum(bufs[k], i + k)

        pltpu.sync_copy(out_v, out_hbm.at[pl.ds(base, PER_W)])

    return kern(x_flat, table)


def _linear_tc(pooled, w, b2):
    def kern(p_ref, w_ref, b_ref, o_ref):
        o_ref[...] = lax.dot_general(
            p_ref[...], w_ref[...], (((1,), (1,)), ((), ())),
            preferred_element_type=jnp.float32) + b_ref[...]

    return pl.pallas_call(
        kern,
        out_shape=jax.ShapeDtypeStruct((B, 2), jnp.float32),
    )(pooled, w, b2)


def kernel(x, embedding, fc_w, fc_b):
    table = _repack_tc(embedding.T).reshape(VOCAB_PAD, D)
    pooled = _pooled_sc(_remap_idx(x), table)
    return _linear_tc(pooled, fc_w, fc_b.reshape(1, 2))
